# baseline (device time: 25903 ns/iter reference)
import jax
import jax.numpy as jnp
from jax import lax
from jax.experimental import pallas as pl
from jax.experimental.pallas import tpu as pltpu

N_DEV = 4
B = 256


def kernel(x):
    m, n = x.shape
    nblk = m // B

    def body(x_hbm, out_hbm, xbuf, csbuf, obuf, comm_ref,
             rsems, wsems, send_sems, recv_sems):
        my = lax.axis_index("i")

        barrier_sem = pltpu.get_barrier_semaphore()
        for k in range(1, N_DEV):
            pl.semaphore_signal(
                barrier_sem, inc=1,
                device_id=(lax.rem(my + k, N_DEV),),
                device_id_type=pl.DeviceIdType.MESH,
            )

        r = lax.broadcasted_iota(jnp.int32, (B, B), 0)
        c = lax.broadcasted_iota(jnp.int32, (B, B), 1)
        L = (r >= c).astype(jnp.bfloat16)

        def read(b, slot):
            return pltpu.make_async_copy(
                x_hbm.at[pl.ds(b * B, B), :], xbuf.at[slot], rsems.at[slot])

        pending = [read(0, 0), None]
        pending[0].start()
        carry = jnp.zeros((1, n), jnp.float32)
        for b in range(nblk):
            slot = b % 2
            if b + 1 < nblk:
                nxt = read(b + 1, 1 - slot)
                nxt.start()
                pending[1 - slot] = nxt
            pending[slot].wait()
            xb = xbuf[slot, :, :].astype(jnp.bfloat16)
            cs = jnp.dot(L, xb, preferred_element_type=jnp.float32)
            csbuf[b * B:(b + 1) * B, :] = cs + carry
            carry = carry + cs[B - 1:B, :]

        comm_ref[0, :, :] = carry

        pl.semaphore_wait(barrier_sem, N_DEV - 1)

        sends = []
        for k in range(1, N_DEV):
            rdma = pltpu.make_async_remote_copy(
                src_ref=comm_ref.at[0],
                dst_ref=comm_ref.at[N_DEV - k],
                send_sem=send_sems.at[k - 1],
                recv_sem=recv_sems.at[N_DEV - k],
                device_id=(lax.rem(my + k, N_DEV),),
                device_id_type=pl.DeviceIdType.MESH,
            )
            rdma.start()
            sends.append(rdma)
        for j in range(1, N_DEV):
            recv = pltpu.make_async_remote_copy(
                src_ref=comm_ref.at[0],
                dst_ref=comm_ref.at[j],
                send_sem=send_sems.at[0],
                recv_sem=recv_sems.at[j],
                device_id=(my,),
                device_id_type=pl.DeviceIdType.MESH,
            )
            recv.wait_recv()
        for rdma in sends:
            rdma.wait_send()

        tots = comm_ref[:, 0, :]
        j = lax.broadcasted_iota(jnp.int32, (N_DEV, n), 0)
        origin = lax.rem(my + j, N_DEV)
        offset = jnp.sum(jnp.where(origin < my, tots, 0.0), axis=0)[None, :]

        wpend = [None, None]
        for b in range(nblk):
            slot = b % 2
            if wpend[slot] is not None:
                wpend[slot].wait()
            obuf[slot, :, :] = csbuf[b * B:(b + 1) * B, :] + offset
            w = pltpu.make_async_copy(
                obuf.at[slot], out_hbm.at[pl.ds(b * B, B), :], wsems.at[slot])
            w.start()
            wpend[slot] = w
        wpend[0].wait()
        wpend[1].wait()

    return pl.pallas_call(
        body,
        out_shape=jax.ShapeDtypeStruct((m, n), x.dtype),
        in_specs=[pl.BlockSpec(memory_space=pl.ANY)],
        out_specs=pl.BlockSpec(memory_space=pl.ANY),
        scratch_shapes=[
            pltpu.VMEM((2, B, n), x.dtype),
            pltpu.VMEM((m, n), jnp.float32),
            pltpu.VMEM((2, B, n), jnp.float32),
            pltpu.VMEM((N_DEV, 1, n), x.dtype),
            pltpu.SemaphoreType.DMA((2,)),
            pltpu.SemaphoreType.DMA((2,)),
            pltpu.SemaphoreType.DMA((N_DEV - 1,)),
            pltpu.SemaphoreType.DMA((N_DEV,)),
        ],
        compiler_params=pltpu.CompilerParams(collective_id=0),
    )(x)


# device time: 19209 ns/iter; 1.3485x vs baseline; 1.3485x over previous
import jax
import jax.numpy as jnp
from jax import lax
from jax.experimental import pallas as pl
from jax.experimental.pallas import tpu as pltpu

N_DEV = 4
B = 256


def kernel(x):
    m, n = x.shape

    def body(x_ref, out_ref, comm_ref, send_sems, recv_sems):
        my = lax.axis_index("i")

        barrier_sem = pltpu.get_barrier_semaphore()
        for k in range(1, N_DEV):
            pl.semaphore_signal(
                barrier_sem, inc=1,
                device_id=(lax.rem(my + k, N_DEV),),
                device_id_type=pl.DeviceIdType.MESH,
            )
        pl.semaphore_wait(barrier_sem, N_DEV - 1)

        comm_ref[0, :, :] = jnp.sum(x_ref[:, :], axis=0, keepdims=True)

        sends = []
        for k in range(1, N_DEV):
            rdma = pltpu.make_async_remote_copy(
                src_ref=comm_ref.at[0],
                dst_ref=comm_ref.at[N_DEV - k],
                send_sem=send_sems.at[k - 1],
                recv_sem=recv_sems.at[N_DEV - k],
                device_id=(lax.rem(my + k, N_DEV),),
                device_id_type=pl.DeviceIdType.MESH,
            )
            rdma.start()
            sends.append(rdma)

        r = lax.broadcasted_iota(jnp.int32, (B, B), 0)
        c = lax.broadcasted_iota(jnp.int32, (B, B), 1)
        L = (r >= c).astype(jnp.bfloat16)

        for j in range(1, N_DEV):
            recv = pltpu.make_async_remote_copy(
                src_ref=comm_ref.at[0],
                dst_ref=comm_ref.at[j],
                send_sem=send_sems.at[0],
                recv_sem=recv_sems.at[j],
                device_id=(my,),
                device_id_type=pl.DeviceIdType.MESH,
            )
            recv.wait_recv()
        for rdma in sends:
            rdma.wait_send()

        tots = comm_ref[:, 0, :]
        j = lax.broadcasted_iota(jnp.int32, (N_DEV, n), 0)
        origin = lax.rem(my + j, N_DEV)
        offset = jnp.sum(jnp.where(origin < my, tots, 0.0), axis=0)

        carry = offset[None, :]
        for b in range(m // B):
            xb = x_ref[b * B:(b + 1) * B, :].astype(jnp.bfloat16)
            cs = jnp.dot(L, xb, preferred_element_type=jnp.float32)
            out_ref[b * B:(b + 1) * B, :] = cs + carry
            carry = carry + cs[B - 1:B, :]

    return pl.pallas_call(
        body,
        out_shape=jax.ShapeDtypeStruct((m, n), x.dtype),
        in_specs=[pl.BlockSpec(memory_space=pltpu.VMEM)],
        out_specs=pl.BlockSpec(memory_space=pltpu.VMEM),
        scratch_shapes=[
            pltpu.VMEM((N_DEV, 1, n), x.dtype),
            pltpu.SemaphoreType.DMA((N_DEV - 1,)),
            pltpu.SemaphoreType.DMA((N_DEV,)),
        ],
        compiler_params=pltpu.CompilerParams(collective_id=0),
    )(x)
